# Initial kernel scaffold; baseline (speedup 1.0000x reference)
import jax
import jax.numpy as jnp
from jax.experimental import pallas as pl


def kernel(seq, idx, bond_feats, dist_matrix, emb_res_W, emb_atom_W):
    raise NotImplementedError("write your pallas kernel here")


# speedup vs baseline: 6.2532x; 6.2532x over previous
import functools

import jax
import jax.numpy as jnp
from jax import lax
from jax.experimental import pallas as pl
from jax.experimental.pallas import tpu as pltpu
from jax.experimental.pallas import tpu_sc as plsc

L = 1024
D = 64
NRES = 66
NATOM = 10
NC = 2
NS = 16
NW = NC * NS
ROWS_PER_W = L // NW
LANES = 16


def _table_body(res_ref, atom_ref, t_ref):
    t_ref[...] = res_ref[...][:, None, :] + atom_ref[...][None, :, :]


def _build_table(emb_res_W, emb_atom_W):
    t3 = pl.pallas_call(
        _table_body,
        out_shape=jax.ShapeDtypeStruct((NRES, NATOM, D), jnp.float32),
    )(emb_res_W, emb_atom_W)
    return t3.reshape(NRES * NATOM, D)


def _sc_body(table_hbm, idx_hbm, msk_hbm, idxb_hbm, mskb_hbm, dist_hbm, out_hbm,
             idx_v, msk_v, idxb_v, mskb_v, dist_v, cid_v, rows_v, sem):
    wid = lax.axis_index("s") * NC + lax.axis_index("c")
    row0 = wid * ROWS_PER_W
    pltpu.sync_copy(idx_hbm, idx_v)
    pltpu.sync_copy(msk_hbm, msk_v)
    pltpu.sync_copy(idxb_hbm.at[pl.ds(row0, ROWS_PER_W)], idxb_v)
    pltpu.sync_copy(mskb_hbm.at[pl.ds(row0, ROWS_PER_W)], mskb_v)

    def row_step(r, carry):
        row = row0 + r
        pltpu.sync_copy(dist_hbm.at[row], dist_v)
        idx_i = idxb_v[r]
        msk_i = mskb_v[r]

        for s in range(L // LANES):
            cols = pl.ds(s * LANES, LANES)
            idx_j = idx_v[cols]
            msk_j = msk_v[cols]
            d = dist_v[cols]
            res = jnp.clip(idx_j - idx_i, -32, 33) + 32
            res = jnp.where(msk_i + msk_j == 0, res, 65)
            dc = jnp.clip(d, 0.0, 9.0)
            tr = dc.astype(jnp.int32)
            ia = jnp.where(dc > tr.astype(jnp.float32), tr + 1, tr)
            ia = jnp.where(msk_i + msk_j == 2, ia, 9)
            cid_v[s >> 3, pl.ds((s & 7) * LANES, LANES)] = res * NATOM + ia

        for j in range(8):
            pltpu.async_copy(table_hbm.at[cid_v.at[j]],
                             rows_v.at[pl.ds(j * 128, 128)], sem).wait()
        pltpu.sync_copy(rows_v, out_hbm.at[pl.ds(row * L, L)])
        return carry

    lax.fori_loop(0, ROWS_PER_W, row_step, 0)


def kernel(seq, idx, bond_feats, dist_matrix, emb_res_W, emb_atom_W):
    del bond_feats
    table = _build_table(emb_res_W, emb_atom_W)
    idx32 = idx[0].astype(jnp.int32)
    msk32 = (seq[0] >= 32).astype(jnp.int32)
    idxb = jnp.tile(idx32[:, None], (1, LANES))
    mskb = jnp.tile(msk32[:, None], (1, LANES))
    dist = dist_matrix[0]
    sc = pl.kernel(
        _sc_body,
        out_type=jax.ShapeDtypeStruct((L * L, D), jnp.float32),
        mesh=plsc.VectorSubcoreMesh(core_axis_name="c", subcore_axis_name="s"),
        scratch_types=[
            pltpu.VMEM((L,), jnp.int32),
            pltpu.VMEM((L,), jnp.int32),
            pltpu.VMEM((ROWS_PER_W, LANES), jnp.int32),
            pltpu.VMEM((ROWS_PER_W, LANES), jnp.int32),
            pltpu.VMEM((L,), jnp.float32),
            pltpu.VMEM((8, 128), jnp.int32),
            pltpu.VMEM((L, D), jnp.float32),
            pltpu.SemaphoreType.DMA,
        ],
        compiler_params=pltpu.CompilerParams(use_tc_tiling_on_sc=False),
    )
    out = sc(table, idx32, msk32, idxb, mskb, dist)
    return out.reshape(1, L, L, D)


# speedup vs baseline: 6.2538x; 1.0001x over previous
import functools

import jax
import jax.numpy as jnp
from jax import lax
from jax.experimental import pallas as pl
from jax.experimental.pallas import tpu as pltpu
from jax.experimental.pallas import tpu_sc as plsc

L = 1024
D = 64
NRES = 66
NATOM = 10
NC = 2
NS = 16
NW = NC * NS
ROWS_PER_W = L // NW
LANES = 16


def _table_body(res_ref, atom_ref, t_ref):
    t_ref[...] = res_ref[...][:, None, :] + atom_ref[...][None, :, :]


def _build_table(emb_res_W, emb_atom_W):
    t3 = pl.pallas_call(
        _table_body,
        out_shape=jax.ShapeDtypeStruct((NRES, NATOM, D), jnp.float32),
    )(emb_res_W, emb_atom_W)
    return t3.reshape(NRES * NATOM, D)


CHUNK = 512
NCHUNK = ROWS_PER_W * 2
NGATH = CHUNK // 128


def _sc_body(table_hbm, idx_hbm, msk_hbm, idxb_hbm, mskb_hbm, dist_hbm, out_hbm,
             idx_v, msk_v, idxb_v, mskb_v, dist_v, cid_v, rows_v,
             gsemA, gsemB, osemA, osemB):
    wid = lax.axis_index("s") * NC + lax.axis_index("c")
    row0 = wid * ROWS_PER_W
    pltpu.sync_copy(idx_hbm, idx_v)
    pltpu.sync_copy(msk_hbm, msk_v)
    pltpu.sync_copy(idxb_hbm.at[pl.ds(row0, ROWS_PER_W)], idxb_v)
    pltpu.sync_copy(mskb_hbm.at[pl.ds(row0, ROWS_PER_W)], mskb_v)

    def out_off(c):
        return (row0 + (c >> 1)) * L + (c & 1) * CHUNK

    def compute_cid(c, b):
        r = c >> 1
        colb = (c & 1) * CHUNK
        pltpu.sync_copy(dist_hbm.at[row0 + r, pl.ds(colb, CHUNK)], dist_v.at[b])
        idx_i = idxb_v[r]
        msk_i = mskb_v[r]
        for s in range(CHUNK // LANES):
            col = colb + s * LANES
            idx_j = idx_v[pl.ds(col, LANES)]
            msk_j = msk_v[pl.ds(col, LANES)]
            d = dist_v[b, pl.ds(s * LANES, LANES)]
            res = jnp.clip(idx_j - idx_i, -32, 33) + 32
            res = jnp.where(msk_i + msk_j == 0, res, 65)
            dc = jnp.clip(d, 0.0, 9.0)
            tr = dc.astype(jnp.int32)
            ia = jnp.where(dc > tr.astype(jnp.float32), tr + 1, tr)
            ia = jnp.where(msk_i + msk_j == 2, ia, 9)
            cid_v[b, s >> 3, pl.ds((s & 7) * LANES, LANES)] = res * NATOM + ia

    def fire_gathers(b, sem):
        for j in range(NGATH):
            pltpu.async_copy(table_hbm.at[cid_v.at[b, j]],
                             rows_v.at[b, pl.ds(j * 128, 128)], sem)

    def drain_gathers(b, sem):
        for j in range(NGATH):
            pltpu.make_async_copy(table_hbm.at[cid_v.at[b, j]],
                                  rows_v.at[b, pl.ds(j * 128, 128)], sem).wait()

    def step(i, carry):
        a, bb = 2 * i, 2 * i + 1

        @pl.when(i > 0)
        def _():
            pltpu.make_async_copy(rows_v.at[0],
                                  out_hbm.at[pl.ds(out_off(a - 2), CHUNK)],
                                  osemA).wait()
            pltpu.make_async_copy(rows_v.at[1],
                                  out_hbm.at[pl.ds(out_off(a - 1), CHUNK)],
                                  osemB).wait()

        compute_cid(a, 0)
        fire_gathers(0, gsemA)
        compute_cid(bb, 1)
        fire_gathers(1, gsemB)
        drain_gathers(0, gsemA)
        pltpu.async_copy(rows_v.at[0], out_hbm.at[pl.ds(out_off(a), CHUNK)], osemA)
        drain_gathers(1, gsemB)
        pltpu.async_copy(rows_v.at[1], out_hbm.at[pl.ds(out_off(bb), CHUNK)], osemB)
        return carry

    lax.fori_loop(0, NCHUNK // 2, step, 0)
    pltpu.make_async_copy(rows_v.at[0],
                          out_hbm.at[pl.ds(out_off(NCHUNK - 2), CHUNK)], osemA).wait()
    pltpu.make_async_copy(rows_v.at[1],
                          out_hbm.at[pl.ds(out_off(NCHUNK - 1), CHUNK)], osemB).wait()


def kernel(seq, idx, bond_feats, dist_matrix, emb_res_W, emb_atom_W):
    del bond_feats
    table = _build_table(emb_res_W, emb_atom_W)
    idx32 = idx[0].astype(jnp.int32)
    msk32 = (seq[0] >= 32).astype(jnp.int32)
    idxb = jnp.tile(idx32[:, None], (1, LANES))
    mskb = jnp.tile(msk32[:, None], (1, LANES))
    dist = dist_matrix[0]
    sc = pl.kernel(
        _sc_body,
        out_type=jax.ShapeDtypeStruct((L * L, D), jnp.float32),
        mesh=plsc.VectorSubcoreMesh(core_axis_name="c", subcore_axis_name="s"),
        scratch_types=[
            pltpu.VMEM((L,), jnp.int32),
            pltpu.VMEM((L,), jnp.int32),
            pltpu.VMEM((ROWS_PER_W, LANES), jnp.int32),
            pltpu.VMEM((ROWS_PER_W, LANES), jnp.int32),
            pltpu.VMEM((2, CHUNK), jnp.float32),
            pltpu.VMEM((2, NGATH, 128), jnp.int32),
            pltpu.VMEM((2, CHUNK, D), jnp.float32),
            pltpu.SemaphoreType.DMA,
            pltpu.SemaphoreType.DMA,
            pltpu.SemaphoreType.DMA,
            pltpu.SemaphoreType.DMA,
        ],
        compiler_params=pltpu.CompilerParams(use_tc_tiling_on_sc=False),
    )
    out = sc(table, idx32, msk32, idxb, mskb, dist)
    return out.reshape(1, L, L, D)


# speedup vs baseline: 31.5308x; 5.0419x over previous
import functools

import jax
import jax.numpy as jnp
from jax import lax
from jax.experimental import pallas as pl
from jax.experimental.pallas import tpu as pltpu
from jax.experimental.pallas import tpu_sc as plsc

L = 1024
D = 64
NRES = 66
NATOM = 10
NC = 2
NS = 16
NW = NC * NS
ROWS_PER_W = L // NW
LANES = 16


def _table_body(res_ref, atom_ref, t_ref):
    t_ref[...] = res_ref[...][:, None, :] + atom_ref[...][None, :, :]


def _build_table(emb_res_W, emb_atom_W):
    t3 = pl.pallas_call(
        _table_body,
        out_shape=jax.ShapeDtypeStruct((NRES, NATOM, D), jnp.float32),
    )(emb_res_W, emb_atom_W)
    return t3.reshape(NRES * NATOM, D)


CHUNK = 512
NCHUNK = ROWS_PER_W * 2


def _sc_body(table_hbm, idx_hbm, msk_hbm, idxb_hbm, mskb_hbm, dist_hbm, out_hbm,
             tbl_v, idx_v, msk_v, idxb_v, mskb_v, dist_v, rows_v, osemA, osemB):
    wid = lax.axis_index("s") * NC + lax.axis_index("c")
    row0 = wid * ROWS_PER_W
    pltpu.sync_copy(table_hbm, tbl_v)
    pltpu.sync_copy(idx_hbm, idx_v)
    pltpu.sync_copy(msk_hbm, msk_v)
    pltpu.sync_copy(idxb_hbm.at[pl.ds(row0, ROWS_PER_W)], idxb_v)
    pltpu.sync_copy(mskb_hbm.at[pl.ds(row0, ROWS_PER_W)], mskb_v)
    lane = lax.iota(jnp.int32, LANES)

    def out_off(c):
        return ((row0 + (c >> 1)) * L + (c & 1) * CHUNK) * D

    def chunk_work(c, b):
        r = c >> 1
        colb = (c & 1) * CHUNK
        pltpu.sync_copy(dist_hbm.at[row0 + r, pl.ds(colb, CHUNK)], dist_v.at[b])
        idx_i = idxb_v[r]
        msk_i = mskb_v[r]
        rows_b = rows_v.at[b]

        def group(s, carry):
            col = colb + s * LANES
            idx_j = idx_v[pl.ds(col, LANES)]
            msk_j = msk_v[pl.ds(col, LANES)]
            d = dist_v[b, pl.ds(s * LANES, LANES)]
            res = jnp.clip(idx_j - idx_i, -32, 33) + 32
            res = jnp.where(msk_i + msk_j == 0, res, 65)
            dc = jnp.clip(d, 0.0, 9.0)
            tr = dc.astype(jnp.int32)
            ia = jnp.where(dc > tr.astype(jnp.float32), tr + 1, tr)
            ia = jnp.where(msk_i + msk_j == 2, ia, 9)
            cid = res * NATOM + ia
            src0 = cid * D
            pos0 = s * (LANES * D) + lane * D
            for dd in range(D):
                v = plsc.load_gather(tbl_v, [src0 + dd])
                plsc.store_scatter(rows_b, [pos0 + dd], v)
            return carry

        lax.fori_loop(0, CHUNK // LANES, group, 0)

    def step(i, carry):
        a, bb = 2 * i, 2 * i + 1

        @pl.when(i > 0)
        def _():
            pltpu.make_async_copy(rows_v.at[0],
                                  out_hbm.at[pl.ds(out_off(a - 2), CHUNK * D)],
                                  osemA).wait()
            pltpu.make_async_copy(rows_v.at[1],
                                  out_hbm.at[pl.ds(out_off(a - 1), CHUNK * D)],
                                  osemB).wait()

        chunk_work(a, 0)
        pltpu.async_copy(rows_v.at[0],
                         out_hbm.at[pl.ds(out_off(a), CHUNK * D)], osemA)
        chunk_work(bb, 1)
        pltpu.async_copy(rows_v.at[1],
                         out_hbm.at[pl.ds(out_off(bb), CHUNK * D)], osemB)
        return carry

    lax.fori_loop(0, NCHUNK // 2, step, 0)
    pltpu.make_async_copy(rows_v.at[0],
                          out_hbm.at[pl.ds(out_off(NCHUNK - 2), CHUNK * D)],
                          osemA).wait()
    pltpu.make_async_copy(rows_v.at[1],
                          out_hbm.at[pl.ds(out_off(NCHUNK - 1), CHUNK * D)],
                          osemB).wait()


def kernel(seq, idx, bond_feats, dist_matrix, emb_res_W, emb_atom_W):
    del bond_feats
    table = _build_table(emb_res_W, emb_atom_W).reshape(-1)
    idx32 = idx[0].astype(jnp.int32)
    msk32 = (seq[0] >= 32).astype(jnp.int32)
    idxb = jnp.tile(idx32[:, None], (1, LANES))
    mskb = jnp.tile(msk32[:, None], (1, LANES))
    dist = dist_matrix[0]
    sc = pl.kernel(
        _sc_body,
        out_type=jax.ShapeDtypeStruct((L * L * D,), jnp.float32),
        mesh=plsc.VectorSubcoreMesh(core_axis_name="c", subcore_axis_name="s"),
        scratch_types=[
            pltpu.VMEM((NRES * NATOM * D,), jnp.float32),
            pltpu.VMEM((L,), jnp.int32),
            pltpu.VMEM((L,), jnp.int32),
            pltpu.VMEM((ROWS_PER_W, LANES), jnp.int32),
            pltpu.VMEM((ROWS_PER_W, LANES), jnp.int32),
            pltpu.VMEM((2, CHUNK), jnp.float32),
            pltpu.VMEM((2, CHUNK * D), jnp.float32),
            pltpu.SemaphoreType.DMA,
            pltpu.SemaphoreType.DMA,
        ],
        compiler_params=pltpu.CompilerParams(use_tc_tiling_on_sc=False,
                                             needs_layout_passes=False),
    )
    out = sc(table, idx32, msk32, idxb, mskb, dist)
    return out.reshape(1, L, L, D)


# speedup vs baseline: 45.6428x; 1.4476x over previous
import functools

import jax
import jax.numpy as jnp
from jax import lax
from jax.experimental import pallas as pl
from jax.experimental.pallas import tpu as pltpu
from jax.experimental.pallas import tpu_sc as plsc

L = 1024
D = 64
NRES = 66
NATOM = 10
NC = 2
NS = 16
NW = NC * NS
ROWS_PER_W = L // NW
LANES = 16


def _table_body(res_ref, atom_ref, t_ref):
    t_ref[...] = res_ref[...][:, None, :] + atom_ref[...][None, :, :]


def _build_table(emb_res_W, emb_atom_W):
    t3 = pl.pallas_call(
        _table_body,
        out_shape=jax.ShapeDtypeStruct((NRES, NATOM, D), jnp.float32),
    )(emb_res_W, emb_atom_W)
    return t3.reshape(NRES * NATOM, D)


CHUNK = 512
NCHUNK = ROWS_PER_W * 2


def _sc_body(table_hbm, idx_hbm, msk_hbm, idxb_hbm, mskb_hbm, dist_hbm, out_hbm,
             tbl_v, idx_v, msk_v, idxb_v, mskb_v, dist_v, rows_v, osemA, osemB):
    wid = lax.axis_index("s") * NC + lax.axis_index("c")
    row0 = wid * ROWS_PER_W
    pltpu.sync_copy(table_hbm, tbl_v)
    pltpu.sync_copy(idx_hbm, idx_v)
    pltpu.sync_copy(msk_hbm, msk_v)
    pltpu.sync_copy(idxb_hbm.at[pl.ds(row0, ROWS_PER_W)], idxb_v)
    pltpu.sync_copy(mskb_hbm.at[pl.ds(row0, ROWS_PER_W)], mskb_v)
    lane = lax.iota(jnp.int32, LANES)

    def out_off(c):
        return ((row0 + (c >> 1)) * L + (c & 1) * CHUNK) * D

    def chunk_work(c, b):
        r = c >> 1
        colb = (c & 1) * CHUNK
        pltpu.sync_copy(dist_hbm.at[row0 + r, pl.ds(colb, CHUNK)], dist_v.at[b])
        idx_i = idxb_v[r]
        msk_i = mskb_v[r]
        rows_b = rows_v.at[b]

        @plsc.parallel_loop(0, CHUNK // LANES)
        def group(s):
            col = colb + s * LANES
            idx_j = idx_v[pl.ds(col, LANES)]
            msk_j = msk_v[pl.ds(col, LANES)]
            d = dist_v[b, pl.ds(s * LANES, LANES)]
            res = jnp.clip(idx_j - idx_i, -32, 33) + 32
            res = jnp.where(msk_i + msk_j == 0, res, 65)
            dc = jnp.clip(d, 0.0, 9.0)
            tr = dc.astype(jnp.int32)
            ia = jnp.where(dc > tr.astype(jnp.float32), tr + 1, tr)
            ia = jnp.where(msk_i + msk_j == 2, ia, 9)
            cid = res * NATOM + ia
            src0 = cid * D
            pos0 = s * (LANES * D) + lane * D
            for dd in range(D):
                v = plsc.load_gather(tbl_v, [src0 + dd])
                plsc.store_scatter(rows_b, [pos0 + dd], v)

    def step(i, carry):
        a, bb = 2 * i, 2 * i + 1

        @pl.when(i > 0)
        def _():
            pltpu.make_async_copy(rows_v.at[0],
                                  out_hbm.at[pl.ds(out_off(a - 2), CHUNK * D)],
                                  osemA).wait()
            pltpu.make_async_copy(rows_v.at[1],
                                  out_hbm.at[pl.ds(out_off(a - 1), CHUNK * D)],
                                  osemB).wait()

        chunk_work(a, 0)
        pltpu.async_copy(rows_v.at[0],
                         out_hbm.at[pl.ds(out_off(a), CHUNK * D)], osemA)
        chunk_work(bb, 1)
        pltpu.async_copy(rows_v.at[1],
                         out_hbm.at[pl.ds(out_off(bb), CHUNK * D)], osemB)
        return carry

    lax.fori_loop(0, NCHUNK // 2, step, 0)
    pltpu.make_async_copy(rows_v.at[0],
                          out_hbm.at[pl.ds(out_off(NCHUNK - 2), CHUNK * D)],
                          osemA).wait()
    pltpu.make_async_copy(rows_v.at[1],
                          out_hbm.at[pl.ds(out_off(NCHUNK - 1), CHUNK * D)],
                          osemB).wait()


def kernel(seq, idx, bond_feats, dist_matrix, emb_res_W, emb_atom_W):
    del bond_feats
    table = _build_table(emb_res_W, emb_atom_W).reshape(-1)
    idx32 = idx[0].astype(jnp.int32)
    msk32 = (seq[0] >= 32).astype(jnp.int32)
    idxb = jnp.tile(idx32[:, None], (1, LANES))
    mskb = jnp.tile(msk32[:, None], (1, LANES))
    dist = dist_matrix[0]
    sc = pl.kernel(
        _sc_body,
        out_type=jax.ShapeDtypeStruct((L * L * D,), jnp.float32),
        mesh=plsc.VectorSubcoreMesh(core_axis_name="c", subcore_axis_name="s"),
        scratch_types=[
            pltpu.VMEM((NRES * NATOM * D,), jnp.float32),
            pltpu.VMEM((L,), jnp.int32),
            pltpu.VMEM((L,), jnp.int32),
            pltpu.VMEM((ROWS_PER_W, LANES), jnp.int32),
            pltpu.VMEM((ROWS_PER_W, LANES), jnp.int32),
            pltpu.VMEM((2, CHUNK), jnp.float32),
            pltpu.VMEM((2, CHUNK * D), jnp.float32),
            pltpu.SemaphoreType.DMA,
            pltpu.SemaphoreType.DMA,
        ],
        compiler_params=pltpu.CompilerParams(use_tc_tiling_on_sc=False,
                                             needs_layout_passes=False),
    )
    out = sc(table, idx32, msk32, idxb, mskb, dist)
    return out.reshape(1, L, L, D)


# speedup vs baseline: 115.7915x; 2.5369x over previous
import functools

import jax
import jax.numpy as jnp
from jax import lax
from jax.experimental import pallas as pl
from jax.experimental.pallas import tpu as pltpu
from jax.experimental.pallas import tpu_sc as plsc

L = 1024
D = 64
NRES = 66
NATOM = 10
NC = 2
NS = 16
NW = NC * NS
ROWS_PER_W = L // NW
LANES = 16


def _table_body(res_ref, atom_ref, t_ref):
    t_ref[...] = res_ref[...][:, None, :] + atom_ref[...][None, :, :]


def _build_table(emb_res_W, emb_atom_W):
    t3 = pl.pallas_call(
        _table_body,
        out_shape=jax.ShapeDtypeStruct((NRES, NATOM, D), jnp.float32),
    )(emb_res_W, emb_atom_W)
    return t3.reshape(NRES * NATOM, D)


CHUNK = 512
NCHUNK = ROWS_PER_W * 2


def _sc_body(table_hbm, idx_hbm, msk_hbm, idxb_hbm, mskb_hbm, dist_hbm, out_hbm,
             tbl_v, idx_v, msk_v, idxb_v, mskb_v, dist_v, base_v, rows_v,
             osemA, osemB):
    wid = lax.axis_index("s") * NC + lax.axis_index("c")
    row0 = wid * ROWS_PER_W
    pltpu.sync_copy(table_hbm, tbl_v)
    pltpu.sync_copy(idx_hbm, idx_v)
    pltpu.sync_copy(msk_hbm, msk_v)
    pltpu.sync_copy(idxb_hbm.at[pl.ds(row0, ROWS_PER_W)], idxb_v)
    pltpu.sync_copy(mskb_hbm.at[pl.ds(row0, ROWS_PER_W)], mskb_v)
    lane = lax.iota(jnp.int32, LANES)

    def out_off(c):
        return ((row0 + (c >> 1)) * L + (c & 1) * CHUNK) * D

    def chunk_work(c, b):
        r = c >> 1
        colb = (c & 1) * CHUNK
        pltpu.sync_copy(dist_hbm.at[row0 + r, pl.ds(colb, CHUNK)], dist_v.at[b])
        idx_i = idxb_v[r]
        msk_i = mskb_v[r]
        rows_b = rows_v.at[b]
        base_b = base_v.at[b]

        @plsc.parallel_loop(0, CHUNK // LANES)
        def group(s):
            col = colb + s * LANES
            idx_j = idx_v[pl.ds(col, LANES)]
            msk_j = msk_v[pl.ds(col, LANES)]
            d = dist_v[b, pl.ds(s * LANES, LANES)]
            res = jnp.clip(idx_j - idx_i, -32, 33) + 32
            res = jnp.where(msk_i + msk_j == 0, res, 65)
            dc = jnp.clip(d, 0.0, 9.0)
            tr = dc.astype(jnp.int32)
            ia = jnp.where(dc > tr.astype(jnp.float32), tr + 1, tr)
            ia = jnp.where(msk_i + msk_j == 2, ia, 9)
            cid = res * NATOM + ia
            base_b[s] = cid * D
            sv = jnp.full((LANES,), s, jnp.int32)
            for p in range(LANES):
                bc = plsc.load_gather(base_b, [sv, jnp.full((LANES,), p, jnp.int32)])
                off = s * (LANES * D) + p * D
                for t in range(D // LANES):
                    v = plsc.load_gather(tbl_v, [bc + (lane + t * LANES)])
                    rows_b[pl.ds(off + t * LANES, LANES)] = v

    def step(i, carry):
        a, bb = 2 * i, 2 * i + 1

        @pl.when(i > 0)
        def _():
            pltpu.make_async_copy(rows_v.at[0],
                                  out_hbm.at[pl.ds(out_off(a - 2), CHUNK * D)],
                                  osemA).wait()
            pltpu.make_async_copy(rows_v.at[1],
                                  out_hbm.at[pl.ds(out_off(a - 1), CHUNK * D)],
                                  osemB).wait()

        chunk_work(a, 0)
        pltpu.async_copy(rows_v.at[0],
                         out_hbm.at[pl.ds(out_off(a), CHUNK * D)], osemA)
        chunk_work(bb, 1)
        pltpu.async_copy(rows_v.at[1],
                         out_hbm.at[pl.ds(out_off(bb), CHUNK * D)], osemB)
        return carry

    lax.fori_loop(0, NCHUNK // 2, step, 0)
    pltpu.make_async_copy(rows_v.at[0],
                          out_hbm.at[pl.ds(out_off(NCHUNK - 2), CHUNK * D)],
                          osemA).wait()
    pltpu.make_async_copy(rows_v.at[1],
                          out_hbm.at[pl.ds(out_off(NCHUNK - 1), CHUNK * D)],
                          osemB).wait()


def kernel(seq, idx, bond_feats, dist_matrix, emb_res_W, emb_atom_W):
    del bond_feats
    table = _build_table(emb_res_W, emb_atom_W).reshape(-1)
    idx32 = idx[0].astype(jnp.int32)
    msk32 = (seq[0] >= 32).astype(jnp.int32)
    idxb = jnp.tile(idx32[:, None], (1, LANES))
    mskb = jnp.tile(msk32[:, None], (1, LANES))
    dist = dist_matrix[0]
    sc = pl.kernel(
        _sc_body,
        out_type=jax.ShapeDtypeStruct((L * L * D,), jnp.float32),
        mesh=plsc.VectorSubcoreMesh(core_axis_name="c", subcore_axis_name="s"),
        scratch_types=[
            pltpu.VMEM((NRES * NATOM * D,), jnp.float32),
            pltpu.VMEM((L,), jnp.int32),
            pltpu.VMEM((L,), jnp.int32),
            pltpu.VMEM((ROWS_PER_W, LANES), jnp.int32),
            pltpu.VMEM((ROWS_PER_W, LANES), jnp.int32),
            pltpu.VMEM((2, CHUNK), jnp.float32),
            pltpu.VMEM((2, CHUNK // LANES, LANES), jnp.int32),
            pltpu.VMEM((2, CHUNK * D), jnp.float32),
            pltpu.SemaphoreType.DMA,
            pltpu.SemaphoreType.DMA,
        ],
        compiler_params=pltpu.CompilerParams(use_tc_tiling_on_sc=False,
                                             needs_layout_passes=False),
    )
    out = sc(table, idx32, msk32, idxb, mskb, dist)
    return out.reshape(1, L, L, D)
